# baseline (speedup 1.0000x reference)
import functools
import math

import jax
import jax.numpy as jnp
from jax import lax
from jax.experimental import pallas as pl
from jax.experimental.pallas import tpu as pltpu
from jax.experimental.pallas import tpu_sc as plsc

D = 32
NC, NS, L = 2, 16, 16
NW = NC * NS
W = 128
NBUF = 4
G = 2

_SCALE = math.sqrt(float(D))


@functools.cache
def _make_detile(S, B):
    assert B % (NW * W * NBUF) == 0
    b_per_w = B // NW
    kpw = b_per_w // W
    nblk = B // W
    mesh = plsc.VectorSubcoreMesh(core_axis_name="c", subcore_axis_name="s")
    ND = 4
    GD = 2

    @functools.partial(
        pl.kernel,
        mesh=mesh,
        out_type=jax.ShapeDtypeStruct((S * nblk, W), jnp.int32),
        scratch_types=[
            pltpu.VMEM((ND, kpw, W), jnp.int32),
            pltpu.SemaphoreType.DMA((ND,)),
            pltpu.SemaphoreType.DMA((ND,)),
        ],
        compiler_params=pltpu.CompilerParams(use_tc_tiling_on_sc=True),
    )
    def detile_kernel(idx_t, out, buf, sem_in, sem_out):
        wid = lax.axis_index("s") * NC + lax.axis_index("c")
        bstripe = wid * b_per_w

        def in_copy(s, sl, k):
            return pltpu.make_async_copy(
                idx_t.at[s, pl.ds(bstripe + k * W, W)], buf.at[sl, k],
                sem_in.at[sl])

        def out_copy(s, sl):
            return pltpu.make_async_copy(
                buf.at[sl], out.at[pl.ds(s * nblk + wid * kpw, kpw)],
                sem_out.at[sl])

        for s in range(GD):
            for k in range(kpw):
                in_copy(s, s % ND, k).start()

        def body(s, carry):
            sl = s % ND
            for k in range(kpw):
                in_copy(s, sl, k).wait()
            out_copy(s, sl).start()
            nxt = s + GD

            @pl.when(nxt < S)
            def _refill():
                @pl.when(nxt >= ND)
                def _drain():
                    out_copy(nxt - ND, nxt % ND).wait()

                for k in range(kpw):
                    in_copy(nxt, nxt % ND, k).start()

            return carry

        lax.fori_loop(0, S, body, 0)

        for s in range(S - ND, S):
            out_copy(s, s % ND).wait()

    return detile_kernel


@functools.cache
def _make_gather(S, B):
    assert B % (NW * W * NBUF) == 0
    b_per_w = B // NW
    n_chunks = S * (b_per_w // W)
    mesh = plsc.VectorSubcoreMesh(core_axis_name="c", subcore_axis_name="s")

    @functools.partial(
        pl.kernel,
        mesh=mesh,
        out_type=jax.ShapeDtypeStruct((S, D, B), jnp.float32),
        scratch_types=[
            pltpu.VMEM((B // NW, S), jnp.int32),
            pltpu.VMEM((NBUF, W), jnp.int32),
            pltpu.VMEM((NBUF, W, D), jnp.float32),
            pltpu.VMEM((NBUF, D, W + 1), jnp.float32),
            pltpu.SemaphoreType.DMA((NBUF,)),
            pltpu.SemaphoreType.DMA((NBUF,)),
        ],
        compiler_params=pltpu.CompilerParams(
            use_tc_tiling_on_sc=False, needs_layout_passes=False),
    )
    def gather_kernel(emb, idx2, out3, blk_v, raw_v, g_v, stg_v,
                      sem_g, sem_o):
        wid = lax.axis_index("s") * NC + lax.axis_index("c")
        bstripe = wid * b_per_w
        iota = lax.iota(jnp.int32, L)

        def prep(ci, sl):
            s = ci // NBUF
            base = (ci % NBUF) * W
            cs = jnp.full((L,), s, jnp.int32)
            for g in range(W // L):
                rows = iota + (base + g * L)
                raw_v[sl, pl.ds(g * L, L)] = plsc.load_gather(
                    blk_v, [rows, cs])

        def gather_copy(sl):
            return pltpu.make_async_copy(
                emb.at[raw_v.at[sl]], g_v.at[sl], sem_g.at[sl])

        def out_copy(ci, sl):
            s = ci // NBUF
            b0 = bstripe + (ci % NBUF) * W
            return pltpu.make_async_copy(
                stg_v.at[sl, :, pl.ds(0, W)], out3.at[s, :, pl.ds(b0, W)],
                sem_o.at[sl])

        pltpu.sync_copy(idx2.at[pl.ds(bstripe, b_per_w)], blk_v)
        for b in range(G):
            prep(b, b)
            gather_copy(b).start()

        r_lo = iota
        r_hi = iota + L

        def chunk_body(ci, carry):
            b = ci % NBUF
            gather_copy(b).wait()

            @pl.when(ci >= NBUF)
            def _drain():
                out_copy(ci - NBUF, b).wait()

            for j in range(W):
                cj = jnp.full((L,), j, jnp.int32)
                v0 = g_v[b, j, pl.ds(0, L)] * _SCALE
                v1 = g_v[b, j, pl.ds(L, L)] * _SCALE
                plsc.store_scatter(stg_v.at[b], [r_lo, cj], v0)
                plsc.store_scatter(stg_v.at[b], [r_hi, cj], v1)

            out_copy(ci, b).start()

            nxt = ci + G
            sp = (b + G) % NBUF

            @pl.when(nxt < n_chunks)
            def _prefetch():
                prep(nxt, sp)
                gather_copy(sp).start()

            return carry

        lax.fori_loop(0, n_chunks, chunk_body, 0)

        for b in range(NBUF):
            out_copy(n_chunks - NBUF + b, b).wait()

    return gather_kernel


def kernel(inputs, emb):
    n, s = inputs.shape
    raw = _make_gather(s, n)(emb, inputs)
    return raw.transpose(2, 0, 1)


# speedup vs baseline: 1.0052x; 1.0052x over previous
import functools
import math

import jax
import jax.numpy as jnp
from jax import lax
from jax.experimental import pallas as pl
from jax.experimental.pallas import tpu as pltpu
from jax.experimental.pallas import tpu_sc as plsc

D = 32
NC, NS, L = 2, 16, 16
NW = NC * NS
W = 128
NBUF = 4
G = 2

_SCALE = math.sqrt(float(D))


@functools.cache
def _make_gather(S, B):
    assert B % (NW * W * NBUF) == 0
    b_per_w = B // NW
    n_chunks = S * (b_per_w // W)
    mesh = plsc.VectorSubcoreMesh(core_axis_name="c", subcore_axis_name="s")

    @functools.partial(
        pl.kernel,
        mesh=mesh,
        out_type=jax.ShapeDtypeStruct((S, D, B), jnp.float32),
        scratch_types=[
            pltpu.VMEM((B // NW, S), jnp.int32),
            pltpu.VMEM((NBUF, W), jnp.int32),
            pltpu.VMEM((NBUF, W, D), jnp.float32),
            pltpu.VMEM((NBUF, D, W + 1), jnp.float32),
            pltpu.SemaphoreType.DMA((NBUF,)),
            pltpu.SemaphoreType.DMA((NBUF,)),
        ],
        compiler_params=pltpu.CompilerParams(
            use_tc_tiling_on_sc=False, needs_layout_passes=False),
    )
    def gather_kernel(emb, idx2, out3, blk_v, raw_v, g_v, stg_v,
                      sem_g, sem_o):
        wid = lax.axis_index("s") * NC + lax.axis_index("c")
        bstripe = wid * b_per_w
        iota = lax.iota(jnp.int32, L)

        def prep(ci, sl):
            s = ci // NBUF
            base = (ci % NBUF) * W
            cs = jnp.full((L,), s, jnp.int32)
            for g in range(W // L):
                rows = iota + (base + g * L)
                raw_v[sl, pl.ds(g * L, L)] = plsc.load_gather(
                    blk_v, [rows, cs])

        def gather_copy(sl):
            return pltpu.make_async_copy(
                emb.at[raw_v.at[sl]], g_v.at[sl], sem_g.at[sl])

        def out_copy(ci, sl):
            s = ci // NBUF
            b0 = bstripe + (ci % NBUF) * W
            return pltpu.make_async_copy(
                stg_v.at[sl, :, pl.ds(0, W)], out3.at[s, :, pl.ds(b0, W)],
                sem_o.at[sl])

        pltpu.sync_copy(idx2.at[pl.ds(bstripe, b_per_w)], blk_v)
        for b in range(G):
            prep(b, b)
            gather_copy(b).start()

        r_lo = iota
        r_hi = iota + L

        def chunk_body(ci, carry):
            b = ci % NBUF
            gather_copy(b).wait()

            @pl.when(ci >= NBUF)
            def _drain():
                out_copy(ci - NBUF, b).wait()

            for j in range(W):
                cj = jnp.full((L,), j, jnp.int32)
                v0 = g_v[b, j, pl.ds(0, L)] * _SCALE
                v1 = g_v[b, j, pl.ds(L, L)] * _SCALE
                plsc.store_scatter(stg_v.at[b], [r_lo, cj], v0)
                plsc.store_scatter(stg_v.at[b], [r_hi, cj], v1)

            out_copy(ci, b).start()

            nxt = ci + G
            sp = (b + G) % NBUF

            @pl.when(nxt < n_chunks)
            def _prefetch():
                prep(nxt, sp)
                gather_copy(sp).start()

            return carry

        lax.fori_loop(0, n_chunks, chunk_body, 0)

        for b in range(NBUF):
            out_copy(n_chunks - NBUF + b, b).wait()

    return gather_kernel


def kernel(inputs, emb):
    n, s = inputs.shape
    raw = _make_gather(s, n)(emb, inputs)
    return raw.transpose(2, 0, 1)
